# initial kernel scaffold (unmeasured)
import jax
import jax.numpy as jnp
from jax import lax
from jax.experimental import pallas as pl
from jax.experimental.pallas import tpu as pltpu


def kernel(
    x,
):
    def body(*refs):
        pass

    out_shape = jax.ShapeDtypeStruct(..., jnp.float32)
    return pl.pallas_call(body, out_shape=out_shape)(...)


# baseline (device time: 206808 ns/iter reference)
import jax
import jax.numpy as jnp
from jax import lax
from jax.experimental import pallas as pl
from jax.experimental.pallas import tpu as pltpu


def kernel(x):
    _, m, n = x.shape
    n_out = n // 2
    half = m // 2

    def body(x_ref, out_ref, recv_ref, send_sems, recv_sems, local_sem):
        my_x = lax.axis_index("x")
        my_y = lax.axis_index("y")
        x_nbr = (1 - my_x, my_y)
        y_nbr = (my_x, 1 - my_y)

        barrier_sem = pltpu.get_barrier_semaphore()
        for nbr in (x_nbr, y_nbr):
            pl.semaphore_signal(
                barrier_sem, inc=1, device_id=nbr,
                device_id_type=pl.DeviceIdType.MESH,
            )
        pl.semaphore_wait(barrier_sem, 2)

        row0 = my_y * half

        rdma_x = pltpu.make_async_remote_copy(
            src_ref=x_ref.at[0, pl.ds(row0, half), pl.ds((1 - my_x) * n_out, n_out)],
            dst_ref=recv_ref.at[pl.ds(row0, half), :],
            send_sem=send_sems.at[0],
            recv_sem=recv_sems.at[0],
            device_id=x_nbr,
            device_id_type=pl.DeviceIdType.MESH,
        )
        rdma_x.start()

        local_cp = pltpu.make_async_copy(
            x_ref.at[0, :, pl.ds(my_x * n_out, n_out)], out_ref, local_sem
        )
        local_cp.start()
        local_cp.wait()

        rdma_x.wait()

        rdma_y = pltpu.make_async_remote_copy(
            src_ref=recv_ref.at[pl.ds(row0, half), :],
            dst_ref=recv_ref.at[pl.ds(row0, half), :],
            send_sem=send_sems.at[1],
            recv_sem=recv_sems.at[1],
            device_id=y_nbr,
            device_id_type=pl.DeviceIdType.MESH,
        )
        rdma_y.start()
        rdma_y.wait()

        out_ref[:, :] = out_ref[:, :] + recv_ref[:, :]

    return pl.pallas_call(
        body,
        out_shape=jax.ShapeDtypeStruct((m, n_out), x.dtype),
        in_specs=[pl.BlockSpec(memory_space=pl.ANY)],
        out_specs=pl.BlockSpec(memory_space=pltpu.VMEM),
        scratch_shapes=[
            pltpu.VMEM((m, n_out), x.dtype),
            pltpu.SemaphoreType.DMA((2,)),
            pltpu.SemaphoreType.DMA((2,)),
            pltpu.SemaphoreType.DMA,
        ],
        compiler_params=pltpu.CompilerParams(
            collective_id=0, vmem_limit_bytes=80 * 1024 * 1024
        ),
    )(x)


# device time: 121507 ns/iter; 1.7020x vs baseline; 1.7020x over previous
import jax
import jax.numpy as jnp
from jax import lax
from jax.experimental import pallas as pl
from jax.experimental.pallas import tpu as pltpu

C = 16


def kernel(x):
    _, m, n = x.shape
    n_out = n // 2
    half = m // 2
    ch = half // C

    def body(x_ref, out_ref, recv_ref,
             send_x_sems, recv_x_sems, send_y_sems, recv_y_sems, local_sem):
        my_x = lax.axis_index("x")
        my_y = lax.axis_index("y")
        x_nbr = (1 - my_x, my_y)
        y_nbr = (my_x, 1 - my_y)

        barrier_sem = pltpu.get_barrier_semaphore()
        for nbr in (x_nbr, y_nbr):
            pl.semaphore_signal(
                barrier_sem, inc=1, device_id=nbr,
                device_id_type=pl.DeviceIdType.MESH,
            )
        pl.semaphore_wait(barrier_sem, 2)

        row0 = my_y * half
        row1 = (1 - my_y) * half

        def rdma_x(c):
            return pltpu.make_async_remote_copy(
                src_ref=x_ref.at[0, pl.ds(row0 + c * ch, ch),
                                 pl.ds((1 - my_x) * n_out, n_out)],
                dst_ref=recv_ref.at[pl.ds(row0 + c * ch, ch), :],
                send_sem=send_x_sems.at[c],
                recv_sem=recv_x_sems.at[c],
                device_id=x_nbr,
                device_id_type=pl.DeviceIdType.MESH,
            )

        def rdma_y(c):
            return pltpu.make_async_remote_copy(
                src_ref=recv_ref.at[pl.ds(row0 + c * ch, ch), :],
                dst_ref=recv_ref.at[pl.ds(row0 + c * ch, ch), :],
                send_sem=send_y_sems.at[c],
                recv_sem=recv_y_sems.at[c],
                device_id=y_nbr,
                device_id_type=pl.DeviceIdType.MESH,
            )

        local_cp = pltpu.make_async_copy(
            x_ref.at[0, :, pl.ds(my_x * n_out, n_out)], out_ref, local_sem
        )
        local_cp.start()

        for c in range(C):
            rdma_x(c).start()

        local_cp.wait()

        for c in range(C):
            d = rdma_x(c)
            d.wait_recv()
            rdma_y(c).start()
            r = row0 + c * ch
            out_ref[pl.ds(r, ch), :] = (
                out_ref[pl.ds(r, ch), :] + recv_ref[pl.ds(r, ch), :]
            )

        for c in range(C):
            rdma_y(c).wait_recv()
            r = row1 + c * ch
            out_ref[pl.ds(r, ch), :] = (
                out_ref[pl.ds(r, ch), :] + recv_ref[pl.ds(r, ch), :]
            )

        for c in range(C):
            rdma_x(c).wait_send()
            rdma_y(c).wait_send()

    return pl.pallas_call(
        body,
        out_shape=jax.ShapeDtypeStruct((m, n_out), x.dtype),
        in_specs=[pl.BlockSpec(memory_space=pl.ANY)],
        out_specs=pl.BlockSpec(memory_space=pltpu.VMEM),
        scratch_shapes=[
            pltpu.VMEM((m, n_out), x.dtype),
            pltpu.SemaphoreType.DMA((C,)),
            pltpu.SemaphoreType.DMA((C,)),
            pltpu.SemaphoreType.DMA((C,)),
            pltpu.SemaphoreType.DMA((C,)),
            pltpu.SemaphoreType.DMA,
        ],
        compiler_params=pltpu.CompilerParams(
            collective_id=0, vmem_limit_bytes=80 * 1024 * 1024
        ),
    )(x)
